# baseline (speedup 1.0000x reference)
import functools

import jax
import jax.numpy as jnp
from jax import lax
from jax.experimental import pallas as pl
from jax.experimental.pallas import tpu as pltpu
from jax.experimental.pallas import tpu_sc as plsc

BEAM = 16
EOS = 2
MINUS_INF = -1e20
NEG_HUGE = -3.0e38
CHUNK = 2048
BIG = 1 << 30
NCPAD = 128


CPB = 7
BLKW = CPB * CHUNK


def _tc_body(nc, ncb, vocab, x_ref, seq_ref, tok_ref, cmadj_ref, stats_ref,
             cms_scr, run_scr):
    cb = pl.program_id(1)
    xb = x_ref[...]
    gcol = jax.lax.broadcasted_iota(jnp.int32, (BEAM, BLKW), 1) + cb * BLKW
    xm = jnp.where(gcol < vocab, xb, NEG_HUGE)
    cms = [jnp.max(xm[:, t * CHUNK:(t + 1) * CHUNK], axis=1)
           for t in range(CPB)]
    cmb = jnp.stack(cms, axis=1)
    mloc = jnp.max(cmb, axis=1, keepdims=True)

    @pl.when(cb == 0)
    def _():
        run_scr[...] = jnp.concatenate(
            [jnp.full((BEAM, 1), NEG_HUGE, jnp.float32),
             jnp.zeros((BEAM, 7), jnp.float32)], axis=1)

    mold = run_scr[:, 0:1]
    sold = run_scr[:, 1:2]
    mnew = jnp.maximum(mold, mloc)
    ssum = jnp.sum(jnp.exp(xm - mnew), axis=1, keepdims=True)
    snew = sold * jnp.exp(mold - mnew) + ssum
    run_scr[:, 0:1] = mnew
    run_scr[:, 1:2] = snew
    cms_scr[cb] = jnp.concatenate(
        [cmb, jnp.full((BEAM, 8 - CPB), NEG_HUGE, jnp.float32)], axis=1)

    @pl.when(cb == ncb - 1)
    def _():
        seqc = seq_ref[0]
        finc = tok_ref[0] == EOS
        m = mnew
        logs = jnp.log(snew)
        cm = jnp.concatenate(
            [cms_scr[i][:, :CPB] for i in range(ncb)], axis=1)[:, :nc]

        adj = ((cm - m) - logs) + seqc
        adj = jnp.where(finc, MINUS_INF, adj)
        col = jax.lax.broadcasted_iota(jnp.int32, (BEAM, nc), 1)
        adj = jnp.where((col == 0) & finc, seqc, adj)
        pad = jnp.full((BEAM, NCPAD - nc), NEG_HUGE, jnp.float32)
        cmadj_ref[0] = jnp.concatenate([adj, pad], axis=1)

        finf = finc.astype(jnp.float32)
        zero = jnp.zeros((BEAM, 4), jnp.float32)
        st = jnp.concatenate([m, logs, seqc, finf, zero], axis=1).T
        stats_ref[0] = jnp.concatenate(
            [st, jnp.zeros((8, 128 - BEAM), jnp.float32)], axis=1)


def _sc_body(nc, vocab, x_hbm, cmadj_hbm, stats_hbm,
             vals_hbm, preds_hbm, toks_hbm,
             cm_v, stats_v, maxv_v, buf_v, ov_v, op_v, ot_v):
    tail = vocab - (nc - 1) * CHUNK
    b = lax.axis_index("c") * 16 + lax.axis_index("s")
    pltpu.sync_copy(cmadj_hbm.at[b], cm_v)
    pltpu.sync_copy(stats_hbm.at[b], stats_v)
    iota16 = lax.iota(jnp.int32, 16)

    def initstep(i, _):
        r = i >> 3
        jj = i & 7
        maxv_v[i] = jnp.max(cm_v[r, pl.ds(jj * 16, 16)])
        return 0

    lax.fori_loop(0, NCPAD, initstep, 0)

    def ext(k, carry):
        ov, op, ot = carry

        def amax(i, c):
            bv, bi = c
            xv = maxv_v[i]
            better = xv > bv
            return (jnp.where(better, xv, bv), jnp.where(better, i, bi))

        v, istar = lax.fori_loop(
            0, NCPAD, amax, (jnp.float32(NEG_HUGE), jnp.int32(0)), unroll=8)
        rstar = istar >> 3
        jjstar = istar & 7
        sl = cm_v[rstar, pl.ds(jjstar * 16, 16)]
        lane = jnp.min(jnp.where(sl == v, iota16, BIG))
        jstar = jjstar * 16 + lane

        rsel = iota16 == rstar
        mr = jnp.sum(jnp.where(rsel, stats_v[0, pl.ds(0, 16)], 0.0))
        lr = jnp.sum(jnp.where(rsel, stats_v[1, pl.ds(0, 16)], 0.0))
        sr = jnp.sum(jnp.where(rsel, stats_v[2, pl.ds(0, 16)], 0.0))
        fin = jnp.sum(jnp.where(rsel, stats_v[3, pl.ds(0, 16)], 0.0)) > 0.5

        def fin_case(_):
            return jnp.int32(EOS), jnp.float32(MINUS_INF)

        def scan_case(_):
            base = jstar * CHUNK
            row = b * 16 + rstar
            row8 = pl.multiple_of((row >> 3) << 3, 8)
            subrow = row & 7

            tailp = -(-tail // 128) * 128

            def copy_tail(_):
                pltpu.sync_copy(
                    x_hbm.at[pl.ds(row8, 8), pl.ds(base, tailp)],
                    buf_v.at[:, pl.ds(0, tailp)])
                return jnp.int32(0)

            def copy_full(_):
                pltpu.sync_copy(
                    x_hbm.at[pl.ds(row8, 8), pl.ds(base, CHUNK)], buf_v)
                return jnp.int32(0)

            lax.cond(jstar == nc - 1, copy_tail, copy_full, 0)

            def sw(c, sc_carry):
                kmin, m2, cnt = sc_carry
                xv = buf_v[subrow, pl.ds(c * 16, 16)]
                gcol = base + c * 16 + iota16
                sc = ((xv - mr) - lr) + sr
                sc = jnp.where(gcol < vocab, sc, NEG_HUGE)
                iseq = sc == v
                kmin = jnp.minimum(kmin, jnp.where(iseq, gcol, BIG))
                cnt = cnt + jnp.where(iseq, 1, 0)
                m2 = jnp.maximum(m2, jnp.where(sc < v, sc, NEG_HUGE))
                return kmin, m2, cnt

            kmin, m2, cnt = lax.fori_loop(
                0, CHUNK // 16, sw,
                (jnp.full((16,), BIG, jnp.int32),
                 jnp.full((16,), NEG_HUGE, jnp.float32),
                 jnp.zeros((16,), jnp.int32)),
                unroll=4)
            cstar = jnp.min(kmin)
            ncnt = jnp.sum(cnt)
            nmax = jnp.where(ncnt > 1, v, jnp.max(m2))
            return cstar, nmax

        cstar, nmax = lax.cond(fin, fin_case, scan_case, 0)

        slot = pl.ds(jjstar * 16, 16)
        upd = jnp.where(iota16 == lane, nmax, cm_v[rstar, slot])
        cm_v[rstar, slot] = upd
        maxv_v[istar] = jnp.max(upd)

        sel = iota16 == k
        ov = jnp.where(sel, v, ov)
        op = jnp.where(sel, rstar, op)
        ot = jnp.where(sel, cstar, ot)
        return ov, op, ot

    ov, op, ot = lax.fori_loop(
        0, BEAM, ext,
        (jnp.zeros((16,), jnp.float32),
         jnp.zeros((16,), jnp.int32),
         jnp.zeros((16,), jnp.int32)))
    ov_v[0, pl.ds(0, 16)] = ov
    op_v[0, pl.ds(0, 16)] = op
    ot_v[0, pl.ds(0, 16)] = ot
    pltpu.sync_copy(ov_v, vals_hbm.at[b])
    pltpu.sync_copy(op_v, preds_hbm.at[b])
    pltpu.sync_copy(ot_v, toks_hbm.at[b])


def kernel(log_probs, sequence_scores, inp_tokens):
    rows, vocab = log_probs.shape
    b = rows // BEAM
    nc = -(-vocab // CHUNK)
    seq3 = sequence_scores.reshape(b, BEAM, 1)
    tok3 = inp_tokens.astype(jnp.int32).reshape(b, BEAM, 1)
    ncb = -(-vocab // BLKW)
    col = pl.BlockSpec((1, BEAM, 1), lambda i, c: (i, 0, 0))
    cmadj, stats = pl.pallas_call(
        functools.partial(_tc_body, nc, ncb, vocab),
        grid=(b, ncb),
        in_specs=[
            pl.BlockSpec((BEAM, BLKW), lambda i, c: (i, c)),
            col, col,
        ],
        out_specs=[
            pl.BlockSpec((1, BEAM, NCPAD), lambda i, c: (i, 0, 0)),
            pl.BlockSpec((1, 8, 128), lambda i, c: (i, 0, 0)),
        ],
        out_shape=[
            jax.ShapeDtypeStruct((b, BEAM, NCPAD), jnp.float32),
            jax.ShapeDtypeStruct((b, 8, 128), jnp.float32),
        ],
        scratch_shapes=[
            pltpu.VMEM((8, BEAM, 8), jnp.float32),
            pltpu.VMEM((BEAM, 8), jnp.float32),
        ],
        compiler_params=pltpu.CompilerParams(
            dimension_semantics=("parallel", "arbitrary")),
    )(log_probs, seq3, tok3)

    mesh = plsc.VectorSubcoreMesh(core_axis_name="c", subcore_axis_name="s")
    sck = functools.partial(
        pl.kernel,
        mesh=mesh,
        compiler_params=pltpu.CompilerParams(needs_layout_passes=False),
        out_type=[
            jax.ShapeDtypeStruct((b, 8, 128), jnp.float32),
            jax.ShapeDtypeStruct((b, 8, 128), jnp.int32),
            jax.ShapeDtypeStruct((b, 8, 128), jnp.int32),
        ],
        scratch_types=[
            pltpu.VMEM((BEAM, NCPAD), jnp.float32),
            pltpu.VMEM((8, 128), jnp.float32),
            pltpu.SMEM((NCPAD,), jnp.float32),
            pltpu.VMEM((8, CHUNK), jnp.float32),
            pltpu.VMEM((8, 128), jnp.float32),
            pltpu.VMEM((8, 128), jnp.int32),
            pltpu.VMEM((8, 128), jnp.int32),
        ],
    )(functools.partial(_sc_body, nc, vocab))
    vals, preds, toks = sck(log_probs, cmadj, stats)
    return vals[:, 0, :BEAM], preds[:, 0, :BEAM], toks[:, 0, :BEAM]


# speedup vs baseline: 1.4021x; 1.4021x over previous
import functools

import jax
import jax.numpy as jnp
from jax import lax
from jax.experimental import pallas as pl
from jax.experimental.pallas import tpu as pltpu
from jax.experimental.pallas import tpu_sc as plsc

BEAM = 16
EOS = 2
MINUS_INF = -1e20
NEG_HUGE = -3.0e38
CHUNK = 2048
BIG = 1 << 30
NCPAD = 128


def _tc_body(nc, vocab, x_ref, seq_ref, tok_ref, cmadj_ref, stats_ref):
    tail = vocab - (nc - 1) * CHUNK
    seqc = seq_ref[0]
    finc = tok_ref[0] == EOS

    cms = []
    for j in range(nc):
        w = CHUNK if j < nc - 1 else tail
        cms.append(jnp.max(x_ref[:, j * CHUNK:j * CHUNK + w], axis=1))
    cm = jnp.stack(cms, axis=1)
    m = jnp.max(cm, axis=1, keepdims=True)
    s = jnp.sum(jnp.exp(x_ref[...] - m), axis=1, keepdims=True)
    logs = jnp.log(s)

    adj = ((cm - m) - logs) + seqc
    adj = jnp.where(finc, MINUS_INF, adj)
    col = jax.lax.broadcasted_iota(jnp.int32, (BEAM, nc), 1)
    adj = jnp.where((col == 0) & finc, seqc, adj)
    pad = jnp.full((BEAM, NCPAD - nc), NEG_HUGE, jnp.float32)
    cmadj_ref[0] = jnp.concatenate([adj, pad], axis=1)

    finf = finc.astype(jnp.float32)
    zero = jnp.zeros((BEAM, 4), jnp.float32)
    st = jnp.concatenate([m, logs, seqc, finf, zero], axis=1).T
    stats_ref[0] = jnp.concatenate(
        [st, jnp.zeros((8, 128 - BEAM), jnp.float32)], axis=1)


def _sc_body(nc, vocab, x_hbm, cmadj_hbm, stats_hbm,
             vals_hbm, preds_hbm, toks_hbm,
             cm_v, stats_v, maxv_v, buf_v, ov_v, op_v, ot_v):
    tail = vocab - (nc - 1) * CHUNK
    b = lax.axis_index("c") * 16 + lax.axis_index("s")
    pltpu.sync_copy(cmadj_hbm.at[b], cm_v)
    pltpu.sync_copy(stats_hbm.at[b], stats_v)
    iota16 = lax.iota(jnp.int32, 16)

    def initstep(i, _):
        r = i >> 3
        jj = i & 7
        maxv_v[i] = jnp.max(cm_v[r, pl.ds(jj * 16, 16)])
        return 0

    lax.fori_loop(0, NCPAD, initstep, 0)

    def ext(k, carry):
        ov, op, ot = carry

        def amax(i, c):
            bv, bi = c
            xv = maxv_v[i]
            better = xv > bv
            return (jnp.where(better, xv, bv), jnp.where(better, i, bi))

        v, istar = lax.fori_loop(
            0, NCPAD, amax, (jnp.float32(NEG_HUGE), jnp.int32(0)), unroll=8)
        rstar = istar >> 3
        jjstar = istar & 7
        sl = cm_v[rstar, pl.ds(jjstar * 16, 16)]
        lane = jnp.min(jnp.where(sl == v, iota16, BIG))
        jstar = jjstar * 16 + lane

        rsel = iota16 == rstar
        mr = jnp.sum(jnp.where(rsel, stats_v[0, pl.ds(0, 16)], 0.0))
        lr = jnp.sum(jnp.where(rsel, stats_v[1, pl.ds(0, 16)], 0.0))
        sr = jnp.sum(jnp.where(rsel, stats_v[2, pl.ds(0, 16)], 0.0))
        fin = jnp.sum(jnp.where(rsel, stats_v[3, pl.ds(0, 16)], 0.0)) > 0.5

        def fin_case(_):
            return jnp.int32(EOS), jnp.float32(MINUS_INF)

        def scan_case(_):
            base = jstar * CHUNK
            row = b * 16 + rstar
            row8 = pl.multiple_of((row >> 3) << 3, 8)
            subrow = row & 7

            tailp = -(-tail // 128) * 128

            def copy_tail(_):
                pltpu.sync_copy(
                    x_hbm.at[pl.ds(row8, 8), pl.ds(base, tailp)],
                    buf_v.at[:, pl.ds(0, tailp)])
                return jnp.int32(0)

            def copy_full(_):
                pltpu.sync_copy(
                    x_hbm.at[pl.ds(row8, 8), pl.ds(base, CHUNK)], buf_v)
                return jnp.int32(0)

            lax.cond(jstar == nc - 1, copy_tail, copy_full, 0)

            def sw(c, sc_carry):
                kmin, m2, cnt = sc_carry
                xv = buf_v[subrow, pl.ds(c * 16, 16)]
                gcol = base + c * 16 + iota16
                sc = ((xv - mr) - lr) + sr
                sc = jnp.where(gcol < vocab, sc, NEG_HUGE)
                iseq = sc == v
                kmin = jnp.minimum(kmin, jnp.where(iseq, gcol, BIG))
                cnt = cnt + jnp.where(iseq, 1, 0)
                m2 = jnp.maximum(m2, jnp.where(sc < v, sc, NEG_HUGE))
                return kmin, m2, cnt

            kmin, m2, cnt = lax.fori_loop(
                0, CHUNK // 16, sw,
                (jnp.full((16,), BIG, jnp.int32),
                 jnp.full((16,), NEG_HUGE, jnp.float32),
                 jnp.zeros((16,), jnp.int32)),
                unroll=4)
            cstar = jnp.min(kmin)
            ncnt = jnp.sum(cnt)
            nmax = jnp.where(ncnt > 1, v, jnp.max(m2))
            return cstar, nmax

        cstar, nmax = lax.cond(fin, fin_case, scan_case, 0)

        slot = pl.ds(jjstar * 16, 16)
        upd = jnp.where(iota16 == lane, nmax, cm_v[rstar, slot])
        cm_v[rstar, slot] = upd
        maxv_v[istar] = jnp.max(upd)

        sel = iota16 == k
        ov = jnp.where(sel, v, ov)
        op = jnp.where(sel, rstar, op)
        ot = jnp.where(sel, cstar, ot)
        return ov, op, ot

    ov, op, ot = lax.fori_loop(
        0, BEAM, ext,
        (jnp.zeros((16,), jnp.float32),
         jnp.zeros((16,), jnp.int32),
         jnp.zeros((16,), jnp.int32)))
    ov_v[0, pl.ds(0, 16)] = ov
    op_v[0, pl.ds(0, 16)] = op
    ot_v[0, pl.ds(0, 16)] = ot
    pltpu.sync_copy(ov_v, vals_hbm.at[b])
    pltpu.sync_copy(op_v, preds_hbm.at[b])
    pltpu.sync_copy(ot_v, toks_hbm.at[b])


def kernel(log_probs, sequence_scores, inp_tokens):
    rows, vocab = log_probs.shape
    b = rows // BEAM
    nc = -(-vocab // CHUNK)
    seq3 = sequence_scores.reshape(b, BEAM, 1)
    tok3 = inp_tokens.astype(jnp.int32).reshape(b, BEAM, 1)
    col = pl.BlockSpec((1, BEAM, 1), lambda i: (i, 0, 0))
    cmadj, stats = pl.pallas_call(
        functools.partial(_tc_body, nc, vocab),
        grid=(b,),
        in_specs=[
            pl.BlockSpec((BEAM, vocab), lambda i: (i, 0)),
            col, col,
        ],
        out_specs=[
            pl.BlockSpec((1, BEAM, NCPAD), lambda i: (i, 0, 0)),
            pl.BlockSpec((1, 8, 128), lambda i: (i, 0, 0)),
        ],
        out_shape=[
            jax.ShapeDtypeStruct((b, BEAM, NCPAD), jnp.float32),
            jax.ShapeDtypeStruct((b, 8, 128), jnp.float32),
        ],
        compiler_params=pltpu.CompilerParams(
            dimension_semantics=("arbitrary",)),
    )(log_probs, seq3, tok3)

    mesh = plsc.VectorSubcoreMesh(core_axis_name="c", subcore_axis_name="s")
    sck = functools.partial(
        pl.kernel,
        mesh=mesh,
        compiler_params=pltpu.CompilerParams(needs_layout_passes=False),
        out_type=[
            jax.ShapeDtypeStruct((b, 8, 128), jnp.float32),
            jax.ShapeDtypeStruct((b, 8, 128), jnp.int32),
            jax.ShapeDtypeStruct((b, 8, 128), jnp.int32),
        ],
        scratch_types=[
            pltpu.VMEM((BEAM, NCPAD), jnp.float32),
            pltpu.VMEM((8, 128), jnp.float32),
            pltpu.SMEM((NCPAD,), jnp.float32),
            pltpu.VMEM((8, CHUNK), jnp.float32),
            pltpu.VMEM((8, 128), jnp.float32),
            pltpu.VMEM((8, 128), jnp.int32),
            pltpu.VMEM((8, 128), jnp.int32),
        ],
    )(functools.partial(_sc_body, nc, vocab))
    vals, preds, toks = sck(log_probs, cmadj, stats)
    return vals[:, 0, :BEAM], preds[:, 0, :BEAM], toks[:, 0, :BEAM]


# speedup vs baseline: 1.4212x; 1.0136x over previous
import functools

import jax
import jax.numpy as jnp
from jax import lax
from jax.experimental import pallas as pl
from jax.experimental.pallas import tpu as pltpu
from jax.experimental.pallas import tpu_sc as plsc

BEAM = 16
EOS = 2
MINUS_INF = -1e20
NEG_HUGE = -3.0e38
CHUNK = 2048
BIG = 1 << 30
NCPAD = 128


def _tc_body(nc, vocab, half, x1_ref, x2_ref, seq_ref, tok_ref,
             cmadj_ref, stats_ref):
    tail = vocab - (nc - 1) * CHUNK
    nc1 = half // CHUNK
    w2 = vocab - half
    seqc = seq_ref[0]
    finc = tok_ref[0] == EOS

    cms = []
    for j in range(nc):
        w = CHUNK if j < nc - 1 else tail
        if j < nc1:
            sl = x1_ref[:, j * CHUNK:j * CHUNK + w]
        else:
            lo = j * CHUNK - half
            sl = x2_ref[:, lo:lo + w]
        cms.append(jnp.max(sl, axis=1))
    cm = jnp.stack(cms, axis=1)
    m = jnp.max(cm, axis=1, keepdims=True)
    s = (jnp.sum(jnp.exp(x1_ref[...] - m), axis=1, keepdims=True)
         + jnp.sum(jnp.exp(x2_ref[:, :w2] - m), axis=1, keepdims=True))
    logs = jnp.log(s)

    adj = ((cm - m) - logs) + seqc
    adj = jnp.where(finc, MINUS_INF, adj)
    col = jax.lax.broadcasted_iota(jnp.int32, (BEAM, nc), 1)
    adj = jnp.where((col == 0) & finc, seqc, adj)
    pad = jnp.full((BEAM, NCPAD - nc), NEG_HUGE, jnp.float32)
    cmadj_ref[0] = jnp.concatenate([adj, pad], axis=1)

    finf = finc.astype(jnp.float32)
    zero = jnp.zeros((BEAM, 4), jnp.float32)
    st = jnp.concatenate([m, logs, seqc, finf, zero], axis=1).T
    stats_ref[0] = jnp.concatenate(
        [st, jnp.zeros((8, 128 - BEAM), jnp.float32)], axis=1)


def _sc_body(nc, vocab, x_hbm, cmadj_hbm, stats_hbm,
             vals_hbm, preds_hbm, toks_hbm,
             cm_v, stats_v, maxv_v, buf_v, ov_v, op_v, ot_v):
    tail = vocab - (nc - 1) * CHUNK
    b = lax.axis_index("c") * 16 + lax.axis_index("s")
    pltpu.sync_copy(cmadj_hbm.at[b], cm_v)
    pltpu.sync_copy(stats_hbm.at[b], stats_v)
    iota16 = lax.iota(jnp.int32, 16)

    def initstep(i, _):
        r = i >> 3
        jj = i & 7
        maxv_v[i] = jnp.max(cm_v[r, pl.ds(jj * 16, 16)])
        return 0

    lax.fori_loop(0, NCPAD, initstep, 0)

    def ext(k, carry):
        ov, op, ot = carry

        def amax(i, c):
            bv, bi = c
            xv = maxv_v[i]
            better = xv > bv
            return (jnp.where(better, xv, bv), jnp.where(better, i, bi))

        v, istar = lax.fori_loop(
            0, NCPAD, amax, (jnp.float32(NEG_HUGE), jnp.int32(0)), unroll=8)
        rstar = istar >> 3
        jjstar = istar & 7
        sl = cm_v[rstar, pl.ds(jjstar * 16, 16)]
        lane = jnp.min(jnp.where(sl == v, iota16, BIG))
        jstar = jjstar * 16 + lane

        rsel = iota16 == rstar
        mr = jnp.sum(jnp.where(rsel, stats_v[0, pl.ds(0, 16)], 0.0))
        lr = jnp.sum(jnp.where(rsel, stats_v[1, pl.ds(0, 16)], 0.0))
        sr = jnp.sum(jnp.where(rsel, stats_v[2, pl.ds(0, 16)], 0.0))
        fin = jnp.sum(jnp.where(rsel, stats_v[3, pl.ds(0, 16)], 0.0)) > 0.5

        def fin_case(_):
            return jnp.int32(EOS), jnp.float32(MINUS_INF)

        def scan_case(_):
            base = jstar * CHUNK
            row = b * 16 + rstar
            row8 = pl.multiple_of((row >> 3) << 3, 8)
            subrow = row & 7

            tailp = -(-tail // 128) * 128

            def copy_tail(_):
                pltpu.sync_copy(
                    x_hbm.at[pl.ds(row8, 8), pl.ds(base, tailp)],
                    buf_v.at[:, pl.ds(0, tailp)])
                return jnp.int32(0)

            def copy_full(_):
                pltpu.sync_copy(
                    x_hbm.at[pl.ds(row8, 8), pl.ds(base, CHUNK)], buf_v)
                return jnp.int32(0)

            lax.cond(jstar == nc - 1, copy_tail, copy_full, 0)

            def sw(c, sc_carry):
                kmin, m2, cnt = sc_carry
                xv = buf_v[subrow, pl.ds(c * 16, 16)]
                gcol = base + c * 16 + iota16
                sc = ((xv - mr) - lr) + sr
                sc = jnp.where(gcol < vocab, sc, NEG_HUGE)
                iseq = sc == v
                kmin = jnp.minimum(kmin, jnp.where(iseq, gcol, BIG))
                cnt = cnt + jnp.where(iseq, 1, 0)
                m2 = jnp.maximum(m2, jnp.where(sc < v, sc, NEG_HUGE))
                return kmin, m2, cnt

            kmin, m2, cnt = lax.fori_loop(
                0, CHUNK // 16, sw,
                (jnp.full((16,), BIG, jnp.int32),
                 jnp.full((16,), NEG_HUGE, jnp.float32),
                 jnp.zeros((16,), jnp.int32)),
                unroll=4)
            cstar = jnp.min(kmin)
            ncnt = jnp.sum(cnt)
            nmax = jnp.where(ncnt > 1, v, jnp.max(m2))
            return cstar, nmax

        cstar, nmax = lax.cond(fin, fin_case, scan_case, 0)

        slot = pl.ds(jjstar * 16, 16)
        upd = jnp.where(iota16 == lane, nmax, cm_v[rstar, slot])
        cm_v[rstar, slot] = upd
        maxv_v[istar] = jnp.max(upd)

        sel = iota16 == k
        ov = jnp.where(sel, v, ov)
        op = jnp.where(sel, rstar, op)
        ot = jnp.where(sel, cstar, ot)
        return ov, op, ot

    ov, op, ot = lax.fori_loop(
        0, BEAM, ext,
        (jnp.zeros((16,), jnp.float32),
         jnp.zeros((16,), jnp.int32),
         jnp.zeros((16,), jnp.int32)))
    ov_v[0, pl.ds(0, 16)] = ov
    op_v[0, pl.ds(0, 16)] = op
    ot_v[0, pl.ds(0, 16)] = ot
    pltpu.sync_copy(ov_v, vals_hbm.at[b])
    pltpu.sync_copy(op_v, preds_hbm.at[b])
    pltpu.sync_copy(ot_v, toks_hbm.at[b])


def kernel(log_probs, sequence_scores, inp_tokens):
    rows, vocab = log_probs.shape
    b = rows // BEAM
    nc = -(-vocab // CHUNK)
    seq3 = sequence_scores.reshape(b, BEAM, 1)
    tok3 = inp_tokens.astype(jnp.int32).reshape(b, BEAM, 1)
    col = pl.BlockSpec((1, BEAM, 1), lambda i: (i, 0, 0))
    half = (nc // 2 + 1) * CHUNK
    cmadj, stats = pl.pallas_call(
        functools.partial(_tc_body, nc, vocab, half),
        grid=(b,),
        in_specs=[
            pl.BlockSpec((BEAM, half), lambda i: (i, 0)),
            pl.BlockSpec((BEAM, half), lambda i: (i, 1)),
            col, col,
        ],
        out_specs=[
            pl.BlockSpec((1, BEAM, NCPAD), lambda i: (i, 0, 0)),
            pl.BlockSpec((1, 8, 128), lambda i: (i, 0, 0)),
        ],
        out_shape=[
            jax.ShapeDtypeStruct((b, BEAM, NCPAD), jnp.float32),
            jax.ShapeDtypeStruct((b, 8, 128), jnp.float32),
        ],
        compiler_params=pltpu.CompilerParams(
            dimension_semantics=("arbitrary",)),
    )(log_probs, log_probs, seq3, tok3)

    mesh = plsc.VectorSubcoreMesh(core_axis_name="c", subcore_axis_name="s")
    sck = functools.partial(
        pl.kernel,
        mesh=mesh,
        compiler_params=pltpu.CompilerParams(needs_layout_passes=False),
        out_type=[
            jax.ShapeDtypeStruct((b, 8, 128), jnp.float32),
            jax.ShapeDtypeStruct((b, 8, 128), jnp.int32),
            jax.ShapeDtypeStruct((b, 8, 128), jnp.int32),
        ],
        scratch_types=[
            pltpu.VMEM((BEAM, NCPAD), jnp.float32),
            pltpu.VMEM((8, 128), jnp.float32),
            pltpu.SMEM((NCPAD,), jnp.float32),
            pltpu.VMEM((8, CHUNK), jnp.float32),
            pltpu.VMEM((8, 128), jnp.float32),
            pltpu.VMEM((8, 128), jnp.int32),
            pltpu.VMEM((8, 128), jnp.int32),
        ],
    )(functools.partial(_sc_body, nc, vocab))
    vals, preds, toks = sck(log_probs, cmadj, stats)
    return vals[:, 0, :BEAM], preds[:, 0, :BEAM], toks[:, 0, :BEAM]


# speedup vs baseline: 1.4315x; 1.0073x over previous
import functools

import jax
import jax.numpy as jnp
from jax import lax
from jax.experimental import pallas as pl
from jax.experimental.pallas import tpu as pltpu
from jax.experimental.pallas import tpu_sc as plsc

BEAM = 16
EOS = 2
MINUS_INF = -1e20
NEG_HUGE = -3.0e38
CHUNK = 2048
BIG = 1 << 30
NCPAD = 128


def _tc_body(nc, vocab, qw, x1_ref, x2_ref, x3_ref, x4_ref, seq_ref,
             tok_ref, cmadj_ref, stats_ref):
    tail = vocab - (nc - 1) * CHUNK
    ncq = qw // CHUNK
    seqc = seq_ref[0]
    finc = tok_ref[0] == EOS
    refs = [x1_ref, x2_ref, x3_ref, x4_ref]

    cms = []
    for j in range(nc):
        w = CHUNK if j < nc - 1 else tail
        q = j // ncq
        lo = (j - q * ncq) * CHUNK
        cms.append(jnp.max(refs[q][:, lo:lo + w], axis=1))
    cm = jnp.stack(cms, axis=1)
    m = jnp.max(cm, axis=1, keepdims=True)
    s = jnp.zeros((BEAM, 1), jnp.float32)
    for q in range(4):
        wq = min(qw, vocab - q * qw)
        s = s + jnp.sum(jnp.exp(refs[q][:, :wq] - m), axis=1, keepdims=True)
    logs = jnp.log(s)

    adj = ((cm - m) - logs) + seqc
    adj = jnp.where(finc, MINUS_INF, adj)
    col = jax.lax.broadcasted_iota(jnp.int32, (BEAM, nc), 1)
    adj = jnp.where((col == 0) & finc, seqc, adj)
    pad = jnp.full((BEAM, NCPAD - nc), NEG_HUGE, jnp.float32)
    cmadj_ref[0] = jnp.concatenate([adj, pad], axis=1)

    finf = finc.astype(jnp.float32)
    zero = jnp.zeros((BEAM, 4), jnp.float32)
    st = jnp.concatenate([m, logs, seqc, finf, zero], axis=1).T
    stats_ref[0] = jnp.concatenate(
        [st, jnp.zeros((8, 128 - BEAM), jnp.float32)], axis=1)


def _sc_body(nc, vocab, x_hbm, cmadj_hbm, stats_hbm,
             vals_hbm, preds_hbm, toks_hbm,
             cm_v, stats_v, maxv_v, buf_v, ov_v, op_v, ot_v):
    tail = vocab - (nc - 1) * CHUNK
    b = lax.axis_index("c") * 16 + lax.axis_index("s")
    pltpu.sync_copy(cmadj_hbm.at[b], cm_v)
    pltpu.sync_copy(stats_hbm.at[b], stats_v)
    iota16 = lax.iota(jnp.int32, 16)

    def initstep(i, _):
        r = i >> 3
        jj = i & 7
        maxv_v[i] = jnp.max(cm_v[r, pl.ds(jj * 16, 16)])
        return 0

    lax.fori_loop(0, NCPAD, initstep, 0)

    def ext(k, carry):
        ov, op, ot = carry

        def amax(i, c):
            bv, bi = c
            xv = maxv_v[i]
            better = xv > bv
            return (jnp.where(better, xv, bv), jnp.where(better, i, bi))

        v, istar = lax.fori_loop(
            0, NCPAD, amax, (jnp.float32(NEG_HUGE), jnp.int32(0)), unroll=8)
        rstar = istar >> 3
        jjstar = istar & 7
        sl = cm_v[rstar, pl.ds(jjstar * 16, 16)]
        lane = jnp.min(jnp.where(sl == v, iota16, BIG))
        jstar = jjstar * 16 + lane

        rsel = iota16 == rstar
        mr = jnp.sum(jnp.where(rsel, stats_v[0, pl.ds(0, 16)], 0.0))
        lr = jnp.sum(jnp.where(rsel, stats_v[1, pl.ds(0, 16)], 0.0))
        sr = jnp.sum(jnp.where(rsel, stats_v[2, pl.ds(0, 16)], 0.0))
        fin = jnp.sum(jnp.where(rsel, stats_v[3, pl.ds(0, 16)], 0.0)) > 0.5

        def fin_case(_):
            return jnp.int32(EOS), jnp.float32(MINUS_INF)

        def scan_case(_):
            base = jstar * CHUNK
            row = b * 16 + rstar
            row8 = pl.multiple_of((row >> 3) << 3, 8)
            subrow = row & 7

            tailp = -(-tail // 128) * 128

            def copy_tail(_):
                pltpu.sync_copy(
                    x_hbm.at[pl.ds(row8, 8), pl.ds(base, tailp)],
                    buf_v.at[:, pl.ds(0, tailp)])
                return jnp.int32(0)

            def copy_full(_):
                pltpu.sync_copy(
                    x_hbm.at[pl.ds(row8, 8), pl.ds(base, CHUNK)], buf_v)
                return jnp.int32(0)

            lax.cond(jstar == nc - 1, copy_tail, copy_full, 0)

            def sw(c, sc_carry):
                kmin, m2, cnt = sc_carry
                xv = buf_v[subrow, pl.ds(c * 16, 16)]
                gcol = base + c * 16 + iota16
                sc = ((xv - mr) - lr) + sr
                sc = jnp.where(gcol < vocab, sc, NEG_HUGE)
                iseq = sc == v
                kmin = jnp.minimum(kmin, jnp.where(iseq, gcol, BIG))
                cnt = cnt + jnp.where(iseq, 1, 0)
                m2 = jnp.maximum(m2, jnp.where(sc < v, sc, NEG_HUGE))
                return kmin, m2, cnt

            kmin, m2, cnt = lax.fori_loop(
                0, CHUNK // 16, sw,
                (jnp.full((16,), BIG, jnp.int32),
                 jnp.full((16,), NEG_HUGE, jnp.float32),
                 jnp.zeros((16,), jnp.int32)),
                unroll=4)
            cstar = jnp.min(kmin)
            ncnt = jnp.sum(cnt)
            nmax = jnp.where(ncnt > 1, v, jnp.max(m2))
            return cstar, nmax

        cstar, nmax = lax.cond(fin, fin_case, scan_case, 0)

        slot = pl.ds(jjstar * 16, 16)
        upd = jnp.where(iota16 == lane, nmax, cm_v[rstar, slot])
        cm_v[rstar, slot] = upd
        maxv_v[istar] = jnp.max(upd)

        sel = iota16 == k
        ov = jnp.where(sel, v, ov)
        op = jnp.where(sel, rstar, op)
        ot = jnp.where(sel, cstar, ot)
        return ov, op, ot

    ov, op, ot = lax.fori_loop(
        0, BEAM, ext,
        (jnp.zeros((16,), jnp.float32),
         jnp.zeros((16,), jnp.int32),
         jnp.zeros((16,), jnp.int32)))
    ov_v[0, pl.ds(0, 16)] = ov
    op_v[0, pl.ds(0, 16)] = op
    ot_v[0, pl.ds(0, 16)] = ot
    pltpu.sync_copy(ov_v, vals_hbm.at[b])
    pltpu.sync_copy(op_v, preds_hbm.at[b])
    pltpu.sync_copy(ot_v, toks_hbm.at[b])


def kernel(log_probs, sequence_scores, inp_tokens):
    rows, vocab = log_probs.shape
    b = rows // BEAM
    nc = -(-vocab // CHUNK)
    seq3 = sequence_scores.reshape(b, BEAM, 1)
    tok3 = inp_tokens.astype(jnp.int32).reshape(b, BEAM, 1)
    col = pl.BlockSpec((1, BEAM, 1), lambda i: (i, 0, 0))
    qw = 13 * CHUNK
    xspec = lambda q: pl.BlockSpec((BEAM, qw), lambda i, _q=q: (i, _q))
    cmadj, stats = pl.pallas_call(
        functools.partial(_tc_body, nc, vocab, qw),
        grid=(b,),
        in_specs=[
            xspec(0), xspec(1), xspec(2), xspec(3),
            col, col,
        ],
        out_specs=[
            pl.BlockSpec((1, BEAM, NCPAD), lambda i: (i, 0, 0)),
            pl.BlockSpec((1, 8, 128), lambda i: (i, 0, 0)),
        ],
        out_shape=[
            jax.ShapeDtypeStruct((b, BEAM, NCPAD), jnp.float32),
            jax.ShapeDtypeStruct((b, 8, 128), jnp.float32),
        ],
        compiler_params=pltpu.CompilerParams(
            dimension_semantics=("arbitrary",)),
    )(log_probs, log_probs, log_probs, log_probs, seq3, tok3)

    mesh = plsc.VectorSubcoreMesh(core_axis_name="c", subcore_axis_name="s")
    sck = functools.partial(
        pl.kernel,
        mesh=mesh,
        compiler_params=pltpu.CompilerParams(needs_layout_passes=False),
        out_type=[
            jax.ShapeDtypeStruct((b, 8, 128), jnp.float32),
            jax.ShapeDtypeStruct((b, 8, 128), jnp.int32),
            jax.ShapeDtypeStruct((b, 8, 128), jnp.int32),
        ],
        scratch_types=[
            pltpu.VMEM((BEAM, NCPAD), jnp.float32),
            pltpu.VMEM((8, 128), jnp.float32),
            pltpu.SMEM((NCPAD,), jnp.float32),
            pltpu.VMEM((8, CHUNK), jnp.float32),
            pltpu.VMEM((8, 128), jnp.float32),
            pltpu.VMEM((8, 128), jnp.int32),
            pltpu.VMEM((8, 128), jnp.int32),
        ],
    )(functools.partial(_sc_body, nc, vocab))
    vals, preds, toks = sck(log_probs, cmadj, stats)
    return vals[:, 0, :BEAM], preds[:, 0, :BEAM], toks[:, 0, :BEAM]


# speedup vs baseline: 1.4722x; 1.0284x over previous
import functools

import jax
import jax.numpy as jnp
from jax import lax
from jax.experimental import pallas as pl
from jax.experimental.pallas import tpu as pltpu
from jax.experimental.pallas import tpu_sc as plsc

BEAM = 16
EOS = 2
MINUS_INF = -1e20
NEG_HUGE = -3.0e38
CHUNK = 1024
BIG = 1 << 30
NCPAD = 128


def _tc_body(nc, vocab, qw, x1_ref, x2_ref, x3_ref, x4_ref, seq_ref,
             tok_ref, cmadj_ref, stats_ref):
    tail = vocab - (nc - 1) * CHUNK
    ncq = qw // CHUNK
    seqc = seq_ref[0]
    finc = tok_ref[0] == EOS
    refs = [x1_ref, x2_ref, x3_ref, x4_ref]

    cms = []
    for j in range(nc):
        w = CHUNK if j < nc - 1 else tail
        q = j // ncq
        lo = (j - q * ncq) * CHUNK
        cms.append(jnp.max(refs[q][:, lo:lo + w], axis=1))
    cm = jnp.stack(cms, axis=1)
    m = jnp.max(cm, axis=1, keepdims=True)
    s = jnp.zeros((BEAM, 1), jnp.float32)
    for q in range(4):
        wq = min(qw, vocab - q * qw)
        s = s + jnp.sum(jnp.exp(refs[q][:, :wq] - m), axis=1, keepdims=True)
    logs = jnp.log(s)

    adj = ((cm - m) - logs) + seqc
    adj = jnp.where(finc, MINUS_INF, adj)
    col = jax.lax.broadcasted_iota(jnp.int32, (BEAM, nc), 1)
    adj = jnp.where((col == 0) & finc, seqc, adj)
    pad = jnp.full((BEAM, NCPAD - nc), NEG_HUGE, jnp.float32)
    cmadj_ref[0] = jnp.concatenate([adj, pad], axis=1)

    finf = finc.astype(jnp.float32)
    zero = jnp.zeros((BEAM, 4), jnp.float32)
    st = jnp.concatenate([m, logs, seqc, finf, zero], axis=1).T
    stats_ref[0] = jnp.concatenate(
        [st, jnp.zeros((8, 128 - BEAM), jnp.float32)], axis=1)


def _sc_body(nc, vocab, x_hbm, cmadj_hbm, stats_hbm,
             vals_hbm, preds_hbm, toks_hbm,
             cm_v, stats_v, maxv_v, buf_v, ov_v, op_v, ot_v):
    tail = vocab - (nc - 1) * CHUNK
    b = lax.axis_index("c") * 16 + lax.axis_index("s")
    pltpu.sync_copy(cmadj_hbm.at[b], cm_v)
    pltpu.sync_copy(stats_hbm.at[b], stats_v)
    iota16 = lax.iota(jnp.int32, 16)

    def initstep(i, _):
        r = i >> 3
        jj = i & 7
        maxv_v[i] = jnp.max(cm_v[r, pl.ds(jj * 16, 16)])
        return 0

    lax.fori_loop(0, NCPAD, initstep, 0)

    def ext(k, carry):
        ov, op, ot = carry

        def amax(i, c):
            bv, bi = c
            xv = maxv_v[i]
            better = xv > bv
            return (jnp.where(better, xv, bv), jnp.where(better, i, bi))

        v, istar = lax.fori_loop(
            0, NCPAD, amax, (jnp.float32(NEG_HUGE), jnp.int32(0)), unroll=8)
        rstar = istar >> 3
        jjstar = istar & 7
        sl = cm_v[rstar, pl.ds(jjstar * 16, 16)]
        lane = jnp.min(jnp.where(sl == v, iota16, BIG))
        jstar = jjstar * 16 + lane

        rsel = iota16 == rstar
        mr = jnp.sum(jnp.where(rsel, stats_v[0, pl.ds(0, 16)], 0.0))
        lr = jnp.sum(jnp.where(rsel, stats_v[1, pl.ds(0, 16)], 0.0))
        sr = jnp.sum(jnp.where(rsel, stats_v[2, pl.ds(0, 16)], 0.0))
        fin = jnp.sum(jnp.where(rsel, stats_v[3, pl.ds(0, 16)], 0.0)) > 0.5

        def fin_case(_):
            return jnp.int32(EOS), jnp.float32(MINUS_INF)

        def scan_case(_):
            base = jstar * CHUNK
            row = b * 16 + rstar
            row8 = pl.multiple_of((row >> 3) << 3, 8)
            subrow = row & 7

            tailp = -(-tail // 128) * 128

            def copy_tail(_):
                pltpu.sync_copy(
                    x_hbm.at[pl.ds(row8, 8), pl.ds(base, tailp)],
                    buf_v.at[:, pl.ds(0, tailp)])
                return jnp.int32(0)

            def copy_full(_):
                pltpu.sync_copy(
                    x_hbm.at[pl.ds(row8, 8), pl.ds(base, CHUNK)], buf_v)
                return jnp.int32(0)

            lax.cond(jstar == nc - 1, copy_tail, copy_full, 0)

            def sw(c, sc_carry):
                kmin, m2, cnt = sc_carry
                xv = buf_v[subrow, pl.ds(c * 16, 16)]
                gcol = base + c * 16 + iota16
                sc = ((xv - mr) - lr) + sr
                sc = jnp.where(gcol < vocab, sc, NEG_HUGE)
                iseq = sc == v
                kmin = jnp.minimum(kmin, jnp.where(iseq, gcol, BIG))
                cnt = cnt + jnp.where(iseq, 1, 0)
                m2 = jnp.maximum(m2, jnp.where(sc < v, sc, NEG_HUGE))
                return kmin, m2, cnt

            kmin, m2, cnt = lax.fori_loop(
                0, CHUNK // 16, sw,
                (jnp.full((16,), BIG, jnp.int32),
                 jnp.full((16,), NEG_HUGE, jnp.float32),
                 jnp.zeros((16,), jnp.int32)),
                unroll=4)
            cstar = jnp.min(kmin)
            ncnt = jnp.sum(cnt)
            nmax = jnp.where(ncnt > 1, v, jnp.max(m2))
            return cstar, nmax

        cstar, nmax = lax.cond(fin, fin_case, scan_case, 0)

        slot = pl.ds(jjstar * 16, 16)
        upd = jnp.where(iota16 == lane, nmax, cm_v[rstar, slot])
        cm_v[rstar, slot] = upd
        maxv_v[istar] = jnp.max(upd)

        sel = iota16 == k
        ov = jnp.where(sel, v, ov)
        op = jnp.where(sel, rstar, op)
        ot = jnp.where(sel, cstar, ot)
        return ov, op, ot

    ov, op, ot = lax.fori_loop(
        0, BEAM, ext,
        (jnp.zeros((16,), jnp.float32),
         jnp.zeros((16,), jnp.int32),
         jnp.zeros((16,), jnp.int32)))
    ov_v[0, pl.ds(0, 16)] = ov
    op_v[0, pl.ds(0, 16)] = op
    ot_v[0, pl.ds(0, 16)] = ot
    pltpu.sync_copy(ov_v, vals_hbm.at[b])
    pltpu.sync_copy(op_v, preds_hbm.at[b])
    pltpu.sync_copy(ot_v, toks_hbm.at[b])


def kernel(log_probs, sequence_scores, inp_tokens):
    rows, vocab = log_probs.shape
    b = rows // BEAM
    nc = -(-vocab // CHUNK)
    seq3 = sequence_scores.reshape(b, BEAM, 1)
    tok3 = inp_tokens.astype(jnp.int32).reshape(b, BEAM, 1)
    col = pl.BlockSpec((1, BEAM, 1), lambda i: (i, 0, 0))
    qw = 26624
    xspec = lambda q: pl.BlockSpec((BEAM, qw), lambda i, _q=q: (i, _q))
    cmadj, stats = pl.pallas_call(
        functools.partial(_tc_body, nc, vocab, qw),
        grid=(b,),
        in_specs=[
            xspec(0), xspec(1), xspec(2), xspec(3),
            col, col,
        ],
        out_specs=[
            pl.BlockSpec((1, BEAM, NCPAD), lambda i: (i, 0, 0)),
            pl.BlockSpec((1, 8, 128), lambda i: (i, 0, 0)),
        ],
        out_shape=[
            jax.ShapeDtypeStruct((b, BEAM, NCPAD), jnp.float32),
            jax.ShapeDtypeStruct((b, 8, 128), jnp.float32),
        ],
        compiler_params=pltpu.CompilerParams(
            dimension_semantics=("arbitrary",)),
    )(log_probs, log_probs, log_probs, log_probs, seq3, tok3)

    mesh = plsc.VectorSubcoreMesh(core_axis_name="c", subcore_axis_name="s")
    sck = functools.partial(
        pl.kernel,
        mesh=mesh,
        compiler_params=pltpu.CompilerParams(needs_layout_passes=False),
        out_type=[
            jax.ShapeDtypeStruct((b, 8, 128), jnp.float32),
            jax.ShapeDtypeStruct((b, 8, 128), jnp.int32),
            jax.ShapeDtypeStruct((b, 8, 128), jnp.int32),
        ],
        scratch_types=[
            pltpu.VMEM((BEAM, NCPAD), jnp.float32),
            pltpu.VMEM((8, 128), jnp.float32),
            pltpu.SMEM((NCPAD,), jnp.float32),
            pltpu.VMEM((8, CHUNK), jnp.float32),
            pltpu.VMEM((8, 128), jnp.float32),
            pltpu.VMEM((8, 128), jnp.int32),
            pltpu.VMEM((8, 128), jnp.int32),
        ],
    )(functools.partial(_sc_body, nc, vocab))
    vals, preds, toks = sck(log_probs, cmadj, stats)
    return vals[:, 0, :BEAM], preds[:, 0, :BEAM], toks[:, 0, :BEAM]
